# baseline (device time: 215376 ns/iter reference)
import math

import jax
import jax.numpy as jnp
from jax import lax
from jax.experimental import pallas as pl
from jax.experimental.pallas import tpu as pltpu

N_DEV = 8
N_SLOT = 4
N_FRAG = 2
Q_BLK = 512


def kernel(q, k, v):
    S, D = q.shape
    H = S // 2
    HF = H // N_FRAG
    scale = 1.0 / math.sqrt(D)
    n_qblk = S // Q_BLK

    def body(q_ref, k_ref, v_ref, out_ref, kva_ref, kvb_ref, l_ref,
             send_a, recv_a, send_b, recv_b, credit_a, credit_b):
        my = lax.axis_index("i")
        right = (my + 1) % N_DEV
        left = (my + N_DEV - 1) % N_DEV

        barrier_sem = pltpu.get_barrier_semaphore()
        for nbr in (left, right):
            pl.semaphore_signal(
                barrier_sem, inc=1,
                device_id=(nbr,), device_id_type=pl.DeviceIdType.MESH,
            )
        pl.semaphore_wait(barrier_sem, 2)

        def frag_copy(kv, slot, f, sems, recvs, dst):
            return pltpu.make_async_remote_copy(
                src_ref=kv.at[slot, f],
                dst_ref=kv.at[(slot + 1) % N_SLOT, f],
                send_sem=sems.at[slot, f],
                recv_sem=recvs.at[(slot + 1) % N_SLOT, f],
                device_id=(dst,),
                device_id_type=pl.DeviceIdType.MESH,
            )

        for h in range(N_DEV):
            s_slot = h % N_SLOT

            sends = []
            if h < N_DEV - 1 and h >= N_SLOT - 1:
                pl.semaphore_wait(credit_a, 1)
                pl.semaphore_wait(credit_b, 1)
            for f in range(N_FRAG):
                if h == 0:
                    rows = pl.ds(f * HF, HF)
                    kva_ref[0, f, 0] = k_ref[rows, :].astype(jnp.bfloat16)
                    kva_ref[0, f, 1] = v_ref[rows, :].astype(jnp.bfloat16)
                    rows_b = pl.ds(H + f * HF, HF)
                    kvb_ref[0, f, 0] = k_ref[rows_b, :].astype(jnp.bfloat16)
                    kvb_ref[0, f, 1] = v_ref[rows_b, :].astype(jnp.bfloat16)
                else:
                    frag_copy(kva_ref, (h - 1) % N_SLOT, f, send_a, recv_a,
                              right).wait_recv()
                    frag_copy(kvb_ref, (h - 1) % N_SLOT, f, send_b, recv_b,
                              left).wait_recv()
                if h < N_DEV - 1:
                    ra = frag_copy(kva_ref, s_slot, f, send_a, recv_a, right)
                    rb = frag_copy(kvb_ref, s_slot, f, send_b, recv_b, left)
                    ra.start()
                    rb.start()
                    sends.append((ra, rb))

            def qblock(b, _, s_slot=s_slot, h=h):
                rows = pl.ds(b * Q_BLK, Q_BLK)
                qb = (q_ref[rows, :] * scale).astype(jnp.bfloat16)
                acc = None
                lacc = None
                for kv in (kva_ref, kvb_ref):
                    for f in range(N_FRAG):
                        k_h = kv[s_slot, f, 0]
                        v_h = kv[s_slot, f, 1]
                        s = lax.dot_general(
                            qb, k_h, (((1,), (1,)), ((), ())),
                            preferred_element_type=jnp.float32,
                        )
                        p = jnp.exp(s)
                        ls = jnp.sum(p, axis=1, keepdims=True)
                        o = jnp.dot(
                            p.astype(jnp.bfloat16), v_h,
                            preferred_element_type=jnp.float32,
                        )
                        acc = o if acc is None else acc + o
                        lacc = ls if lacc is None else lacc + ls
                if h == 0:
                    out_ref[rows, :] = acc
                    l_ref[rows, :] = lacc
                else:
                    out_ref[rows, :] += acc
                    l_ref[rows, :] += lacc
                return 0

            lax.fori_loop(0, n_qblk, qblock, 0)

            for ra, rb in sends:
                ra.wait_send()
                rb.wait_send()
            if h <= N_DEV - 1 - N_SLOT:
                pl.semaphore_signal(
                    credit_a, inc=1,
                    device_id=(left,), device_id_type=pl.DeviceIdType.MESH,
                )
                pl.semaphore_signal(
                    credit_b, inc=1,
                    device_id=(right,), device_id_type=pl.DeviceIdType.MESH,
                )

        out_ref[...] = out_ref[...] / l_ref[...]

    return pl.pallas_call(
        body,
        out_shape=jax.ShapeDtypeStruct((S, D), jnp.float32),
        in_specs=[pl.BlockSpec(memory_space=pltpu.VMEM)] * 3,
        out_specs=pl.BlockSpec(memory_space=pltpu.VMEM),
        scratch_shapes=[
            pltpu.VMEM((N_SLOT, N_FRAG, 2, HF, D), jnp.bfloat16),
            pltpu.VMEM((N_SLOT, N_FRAG, 2, HF, D), jnp.bfloat16),
            pltpu.VMEM((S, 1), jnp.float32),
            pltpu.SemaphoreType.DMA((N_SLOT, N_FRAG)),
            pltpu.SemaphoreType.DMA((N_SLOT, N_FRAG)),
            pltpu.SemaphoreType.DMA((N_SLOT, N_FRAG)),
            pltpu.SemaphoreType.DMA((N_SLOT, N_FRAG)),
            pltpu.SemaphoreType.REGULAR,
            pltpu.SemaphoreType.REGULAR,
        ],
        compiler_params=pltpu.CompilerParams(collective_id=0),
    )(q, k, v)
